# baseline (speedup 1.0000x reference)
import math

import jax
import jax.numpy as jnp
from jax.experimental import pallas as pl

N = 10000
T = 4
NUM_LAYERS = 2
STEPS = 5
T_DIFF = 1.0
H = 4
DH = 16


def _diffusion_layer(x, src, dst, W, b):
    h = x @ W + b
    deg = jnp.zeros((N,), x.dtype).at[dst].add(1.0)
    nrm = jax.lax.rsqrt(jnp.maximum(deg, 1.0))

    def prop(z):
        msg = z[src] * nrm[src][:, None]
        agg = jnp.zeros_like(z).at[dst].add(msg)
        return agg * nrm[:, None]

    z = h
    dt = T_DIFF / STEPS
    for _ in range(STEPS):
        z = z + dt * (prop(z) - z)
    return jax.nn.relu(z)


def _tail_kernel(x_ref, o_ref):
    o_ref[...] = x_ref[...]


def kernel(node_features, edge_index, times, W_in, b_in, W_diff, b_diff, t2v_w0, t2v_b0, t2v_w, t2v_b, W_time, b_time, W_k, W_v, q_attn, W_o, b_o, W_h1, b_h1, W_h2, b_h2):
    embs = []
    for t in range(T):
        x = node_features[t] @ W_in + b_in
        src = edge_index[t, 0]
        dst = edge_index[t, 1]
        for l in range(NUM_LAYERS):
            x = _diffusion_layer(x, src, dst, W_diff[l], b_diff[l])
        embs.append(x)
    Hs = jnp.stack(embs, axis=0)
    tv = jnp.concatenate([times[:, None] * t2v_w0 + t2v_b0,
                          jnp.sin(times[:, None] * t2v_w + t2v_b)], axis=1)
    te = tv @ W_time + b_time
    h = jnp.transpose(Hs, (1, 0, 2)) + te[None, :, :]
    K = (h @ W_k).reshape(N, T, H, DH)
    V = (h @ W_v).reshape(N, T, H, DH)
    scores = jnp.einsum('nthd,hd->nth', K, q_attn) / math.sqrt(DH)
    attn = jax.nn.softmax(scores, axis=1)
    pooled = jnp.einsum('nth,nthd->nhd', attn, V).reshape(N, D := 64)
    out = pooled @ W_o + b_o
    logits = jax.nn.relu(out @ W_h1 + b_h1) @ W_h2 + b_h2
    return pl.pallas_call(
        _tail_kernel,
        out_shape=jax.ShapeDtypeStruct(logits.shape, logits.dtype),
    )(logits)


# speedup vs baseline: 8.3916x; 8.3916x over previous
import functools
import math

import jax
import jax.numpy as jnp
from jax import lax
from jax.experimental import pallas as pl
from jax.experimental.pallas import tpu as pltpu
from jax.experimental.pallas import tpu_sc as plsc

N = 10000
E = 320000
T = 4
F_IN = 128
D = 64
TD = 16
H = 4
DH = 16
STEPS = 5
DT = 0.2

NPAD = 10240
RPT = NPAD // 16
CHUNK = 160
WIN = 128
NW = 158
EPT = NW * WIN
NDUMP = NPAD - N

f32 = jnp.float32
i32 = jnp.int32

_MESH = plsc.VectorSubcoreMesh(
    core_axis_name="c", subcore_axis_name="s", num_cores=2, num_subcores=16)
_SC_PARAMS = pltpu.CompilerParams(use_tc_tiling_on_sc=False)


def _deg_body(dst_hbm, ones_hbm, zeros_hbm, deg_out, dstv, onesv, degacc):
    c = lax.axis_index("c")
    s = lax.axis_index("s")
    pltpu.sync_copy(ones_hbm, onesv)
    for si in range(2):
        t = 2 * c + si
        w = t * 16 + s
        pltpu.sync_copy(dst_hbm.at[w], dstv)
        pltpu.sync_copy(zeros_hbm, degacc.at[pl.ds(s * RPT, RPT)])
        plsc.subcore_barrier()

        def win(j, carry):
            pltpu.sync_copy(onesv, degacc.at[dstv.at[j]], add=True)
            return carry

        lax.fori_loop(0, NW, win, 0)
        plsc.subcore_barrier()
        pltpu.sync_copy(degacc.at[pl.ds(s * RPT, RPT)],
                        deg_out.at[pl.ds(t * NPAD + s * RPT, RPT)])
        plsc.subcore_barrier()


def _diff_body(y_in, a_hbm, src_hbm, dst_hbm, zeros_hbm, y_out,
               srcv, dstv, g0, accb, yb, ab, acc, sem0):
    c = lax.axis_index("c")
    s = lax.axis_index("s")
    for si in range(2):
        t = 2 * c + si
        w = t * 16 + s
        pltpu.sync_copy(src_hbm.at[w], srcv)
        pltpu.sync_copy(dst_hbm.at[w], dstv)
        if si == 0:
            pltpu.sync_copy(zeros_hbm, acc.at[pl.ds(s * RPT, RPT)])
        base = t * NPAD + s * RPT
        for p in range(STEPS):
            ytab = y_in if p == 0 else y_out
            plsc.subcore_barrier()

            def win(j, carry, ytab=ytab):
                pltpu.async_copy(ytab.at[srcv.at[j]], g0, sem0).wait()
                pltpu.sync_copy(g0, acc.at[dstv.at[j]], add=True)
                return carry

            lax.fori_loop(0, NW, win, 0)
            plsc.subcore_barrier()
            for k in range(4):
                l0 = s * RPT + k * CHUNK
                r0 = base + k * CHUNK
                pltpu.sync_copy(acc.at[pl.ds(l0, CHUNK)], accb)
                pltpu.sync_copy(zeros_hbm.at[pl.ds(0, CHUNK)],
                                acc.at[pl.ds(l0, CHUNK)])
                pltpu.sync_copy(ytab.at[pl.ds(r0, CHUNK)], yb)
                pltpu.sync_copy(a_hbm.at[pl.ds(r0, CHUNK)], ab)

                def upd(r, carry):
                    for cc in range(4):
                        sl = pl.ds(cc * 16, 16)
                        yb[r, sl] = 0.8 * yb[r, sl] + ab[r, sl] * accb[r, sl]
                    return carry

                lax.fori_loop(0, CHUNK, upd, 0)
                pltpu.sync_copy(yb, y_out.at[pl.ds(r0, CHUNK)])


def _sc_degree(dst_g, ones_w, zeros_w8):
    fn = pl.kernel(
        _deg_body,
        jax.ShapeDtypeStruct((T * NPAD, 8), f32),
        mesh=_MESH,
        scratch_types=[
            pltpu.VMEM((NW, WIN), i32),
            pltpu.VMEM((WIN, 8), f32),
            pltpu.VMEM_SHARED((NPAD, 8), f32),
        ],
        compiler_params=_SC_PARAMS,
    )
    return fn(dst_g, ones_w, zeros_w8)


def _sc_diffusion(y0, a_full, src_g, dst_g, zeros_w64):
    fn = pl.kernel(
        _diff_body,
        jax.ShapeDtypeStruct((T * NPAD, D), f32),
        mesh=_MESH,
        scratch_types=[
            pltpu.VMEM((NW, WIN), i32),
            pltpu.VMEM((NW, WIN), i32),
            pltpu.VMEM((WIN, D), f32),
            pltpu.VMEM((CHUNK, D), f32),
            pltpu.VMEM((CHUNK, D), f32),
            pltpu.VMEM((CHUNK, D), f32),
            pltpu.VMEM_SHARED((NPAD, D), f32),
            pltpu.SemaphoreType.DMA,
        ],
        compiler_params=_SC_PARAMS,
    )
    return fn(y0, a_full, src_g, dst_g, zeros_w64)


RBLK = 512


def _p1_body(nf_ref, deg_ref, wi_ref, bi_ref, w0_ref, b0_ref,
             y0_ref, a_ref, nrm_ref, invn_ref):
    deg = jnp.maximum(deg_ref[:, 0:1], 1.0)
    nrm = lax.rsqrt(deg)
    x0 = jnp.dot(nf_ref[...], wi_ref[...],
                 preferred_element_type=f32) + bi_ref[...]
    h = jnp.dot(x0, w0_ref[...], preferred_element_type=f32) + b0_ref[...]
    y0_ref[...] = h * nrm
    a_ref[...] = jnp.broadcast_to(DT * nrm * nrm, (RBLK, D))
    nrm_ref[...] = jnp.broadcast_to(nrm, (RBLK, D))
    invn_ref[...] = jnp.broadcast_to(1.0 / nrm, (RBLK, D))


def _p2_body(y5_ref, nrm_ref, invn_ref, w1_ref, b1_ref, yo_ref):
    x1 = jnp.maximum(y5_ref[...], 0.0) * invn_ref[...]
    h = jnp.dot(x1, w1_ref[...], preferred_element_type=f32) + b1_ref[...]
    yo_ref[...] = h * nrm_ref[...]


def _attn_body(y_ref, invn_ref, te_ref, wk_ref, wv_ref, qf_ref, sel_ref,
               selt_ref, wo_ref, bo_ref, wh1_ref, bh1_ref, wh2_ref, bh2_ref,
               o_ref):
    scale = 1.0 / math.sqrt(DH)
    ss = []
    vs = []
    for t in range(T):
        x_t = jnp.maximum(y_ref[t], 0.0) * invn_ref[t]
        h_t = x_t + te_ref[t:t + 1, :]
        k_t = jnp.dot(h_t, wk_ref[...], preferred_element_type=f32)
        v_t = jnp.dot(h_t, wv_ref[...], preferred_element_type=f32)
        s_t = jnp.dot(k_t * qf_ref[...], sel_ref[...],
                      preferred_element_type=f32) * scale
        ss.append(s_t)
        vs.append(v_t)
    m = jnp.maximum(jnp.maximum(ss[0], ss[1]), jnp.maximum(ss[2], ss[3]))
    es = [jnp.exp(s_t - m) for s_t in ss]
    z = es[0] + es[1] + es[2] + es[3]
    pooled = jnp.zeros_like(vs[0])
    for t in range(T):
        a_t = jnp.dot(es[t] / z, selt_ref[...], preferred_element_type=f32)
        pooled = pooled + a_t * vs[t]
    out = jnp.dot(pooled, wo_ref[...], preferred_element_type=f32) + bo_ref[...]
    hid = jnp.maximum(
        jnp.dot(out, wh1_ref[...], preferred_element_type=f32) + bh1_ref[...],
        0.0)
    o_ref[...] = jnp.dot(hid, wh2_ref[...],
                         preferred_element_type=f32) + bh2_ref[...]


def _tc_prep1(nf_pad, deg8, W_in, b_in, W0, b0):
    nb = (T * NPAD) // RBLK
    full = lambda shape: pl.BlockSpec(shape, lambda i: (0, 0))
    return pl.pallas_call(
        _p1_body,
        grid=(nb,),
        in_specs=[
            pl.BlockSpec((RBLK, F_IN), lambda i: (i, 0)),
            pl.BlockSpec((RBLK, 8), lambda i: (i, 0)),
            full((F_IN, D)), full((1, D)), full((D, D)), full((1, D)),
        ],
        out_specs=[pl.BlockSpec((RBLK, D), lambda i: (i, 0))] * 4,
        out_shape=[jax.ShapeDtypeStruct((T * NPAD, D), f32)] * 4,
    )(nf_pad, deg8, W_in, b_in, W0, b0)


def _tc_prep2(y5, nrm_e, invn_e, W1, b1):
    nb = (T * NPAD) // RBLK
    full = lambda shape: pl.BlockSpec(shape, lambda i: (0, 0))
    return pl.pallas_call(
        _p2_body,
        grid=(nb,),
        in_specs=[pl.BlockSpec((RBLK, D), lambda i: (i, 0))] * 3 +
                 [full((D, D)), full((1, D))],
        out_specs=pl.BlockSpec((RBLK, D), lambda i: (i, 0)),
        out_shape=jax.ShapeDtypeStruct((T * NPAD, D), f32),
    )(y5, nrm_e, invn_e, W1, b1)


def _tc_attn(y5l, invn_e, te, W_k, W_v, qf, sel, selt, W_o, b_o,
             W_h1, b_h1, W_h2p, b_h2p):
    nbb = NPAD // RBLK
    full = lambda shape: pl.BlockSpec(shape, lambda i: (0, 0))
    y4 = y5l.reshape(T, NPAD, D)
    i4 = invn_e.reshape(T, NPAD, D)
    return pl.pallas_call(
        _attn_body,
        grid=(nbb,),
        in_specs=[
            pl.BlockSpec((T, RBLK, D), lambda i: (0, i, 0)),
            pl.BlockSpec((T, RBLK, D), lambda i: (0, i, 0)),
            full((T, D)), full((D, D)), full((D, D)), full((1, D)),
            full((D, H)), full((H, D)), full((D, D)), full((1, D)),
            full((D, D // 2)), full((1, D // 2)),
            full((D // 2, 128)), full((1, 128)),
        ],
        out_specs=pl.BlockSpec((RBLK, 128), lambda i: (i, 0)),
        out_shape=jax.ShapeDtypeStruct((NPAD, 128), f32),
    )(y4, i4, te, W_k, W_v, qf, sel, selt, W_o, b_o, W_h1, b_h1, W_h2p, b_h2p)


def kernel(node_features, edge_index, times, W_in, b_in, W_diff, b_diff,
           t2v_w0, t2v_b0, t2v_w, t2v_b, W_time, b_time, W_k, W_v, q_attn,
           W_o, b_o, W_h1, b_h1, W_h2, b_h2):
    epad = 16 * EPT - E
    pad_rows = (N + (jnp.arange(epad, dtype=i32) % NDUMP)).astype(i32)
    src_slabs = []
    dst_slabs = []
    for t in range(T):
        s_t = jnp.concatenate([edge_index[t, 0], pad_rows]) + t * NPAD
        d_t = jnp.concatenate([edge_index[t, 1], pad_rows])
        src_slabs.append(s_t.reshape(16, NW, WIN))
        dst_slabs.append(d_t.reshape(16, NW, WIN))
    src_g = jnp.stack(src_slabs).reshape(T * 16, NW, WIN).astype(i32)
    dst_g = jnp.stack(dst_slabs).reshape(T * 16, NW, WIN).astype(i32)

    nf_pad = jnp.pad(node_features, ((0, 0), (0, NPAD - N), (0, 0)))
    nf_pad = nf_pad.reshape(T * NPAD, F_IN)

    ones_w = jnp.ones((WIN, 8), f32)
    zeros_w8 = jnp.zeros((RPT, 8), f32)
    zeros_w64 = jnp.zeros((RPT, D), f32)

    tv = jnp.concatenate([times[:, None] * t2v_w0 + t2v_b0,
                          jnp.sin(times[:, None] * t2v_w + t2v_b)], axis=1)
    te = tv @ W_time + b_time

    qf = q_attn.reshape(1, D)
    sel = jnp.repeat(jnp.eye(H, dtype=f32), DH, axis=0)
    selt = sel.T
    W_h2p = jnp.pad(W_h2, ((0, 0), (0, 128 - W_h2.shape[1])))
    b_h2p = jnp.pad(b_h2, (0, 128 - b_h2.shape[0]))

    deg8 = _sc_degree(dst_g, ones_w, zeros_w8)

    y0, a_full, nrm_e, invn_e = _tc_prep1(
        nf_pad, deg8, W_in, b_in.reshape(1, D),
        W_diff[0], b_diff[0].reshape(1, D))

    y5 = _sc_diffusion(y0, a_full, src_g, dst_g, zeros_w64)
    y0b = _tc_prep2(y5, nrm_e, invn_e, W_diff[1], b_diff[1].reshape(1, D))
    y5b = _sc_diffusion(y0b, a_full, src_g, dst_g, zeros_w64)

    logits_pad = _tc_attn(y5b, invn_e, te, W_k, W_v, qf, sel, selt,
                          W_o, b_o.reshape(1, D), W_h1, b_h1.reshape(1, D // 2),
                          W_h2p, b_h2p.reshape(1, 128))
    return logits_pad[:N, :2]


# speedup vs baseline: 10.6864x; 1.2735x over previous
import functools
import math

import jax
import jax.numpy as jnp
from jax import lax
from jax.experimental import pallas as pl
from jax.experimental.pallas import tpu as pltpu
from jax.experimental.pallas import tpu_sc as plsc

N = 10000
E = 320000
T = 4
F_IN = 128
D = 64
TD = 16
H = 4
DH = 16
STEPS = 5
DT = 0.2

NPAD = 10240
RPT = NPAD // 16
CHUNK = 160
WIN = 128
NW = 158
EPT = NW * WIN
NDUMP = NPAD - N

f32 = jnp.float32
i32 = jnp.int32

_MESH = plsc.VectorSubcoreMesh(
    core_axis_name="c", subcore_axis_name="s", num_cores=2, num_subcores=16)
_SC_PARAMS = pltpu.CompilerParams(use_tc_tiling_on_sc=False)


def _deg_body(dst_hbm, ones_hbm, zeros_hbm, deg_out, dstv, onesv, degacc):
    c = lax.axis_index("c")
    s = lax.axis_index("s")
    pltpu.sync_copy(ones_hbm, onesv)
    for si in range(2):
        t = 2 * c + si
        w = t * 16 + s
        pltpu.sync_copy(dst_hbm.at[w], dstv)
        pltpu.sync_copy(zeros_hbm, degacc.at[pl.ds(s * RPT, RPT)])
        plsc.subcore_barrier()

        def win(j, carry):
            pltpu.sync_copy(onesv, degacc.at[dstv.at[j]], add=True)
            return carry

        lax.fori_loop(0, NW, win, 0)
        plsc.subcore_barrier()
        pltpu.sync_copy(degacc.at[pl.ds(s * RPT, RPT)],
                        deg_out.at[pl.ds(t * NPAD + s * RPT, RPT)])
        plsc.subcore_barrier()


def _diff_body(y_in, a_hbm, src_hbm, dst_hbm, zeros_hbm, y_out,
               srcv, dstv, g0, g1, accb, yb, ab, acc, sem0, sem1):
    c = lax.axis_index("c")
    s = lax.axis_index("s")
    for si in range(2):
        t = 2 * c + si
        w = t * 16 + s
        pltpu.sync_copy(src_hbm.at[w], srcv)
        pltpu.sync_copy(dst_hbm.at[w], dstv)
        if si == 0:
            pltpu.sync_copy(zeros_hbm, acc.at[pl.ds(s * RPT, RPT)])
        base = t * NPAD + s * RPT
        for p in range(STEPS):
            ytab = y_in if p == 0 else y_out
            plsc.subcore_barrier()

            pltpu.async_copy(ytab.at[srcv.at[0]], g0, sem0)

            def win(i, carry, ytab=ytab):
                j = 2 * i
                pltpu.make_async_copy(ytab.at[srcv.at[0]], g0, sem0).wait()
                pltpu.async_copy(ytab.at[srcv.at[j + 1]], g1, sem1)
                pltpu.sync_copy(g0, acc.at[dstv.at[j]], add=True)
                pltpu.make_async_copy(ytab.at[srcv.at[0]], g1, sem1).wait()

                @pl.when(j + 2 < NW)
                def _():
                    pltpu.async_copy(ytab.at[srcv.at[j + 2]], g0, sem0)

                pltpu.sync_copy(g1, acc.at[dstv.at[j + 1]], add=True)
                return carry

            lax.fori_loop(0, NW // 2, win, 0)
            plsc.subcore_barrier()
            for k in range(4):
                l0 = s * RPT + k * CHUNK
                r0 = base + k * CHUNK
                pltpu.sync_copy(acc.at[pl.ds(l0, CHUNK)], accb)
                pltpu.sync_copy(zeros_hbm.at[pl.ds(0, CHUNK)],
                                acc.at[pl.ds(l0, CHUNK)])
                pltpu.sync_copy(ytab.at[pl.ds(r0, CHUNK)], yb)
                pltpu.sync_copy(a_hbm.at[pl.ds(r0, CHUNK)], ab)

                def upd(r, carry):
                    for cc in range(4):
                        sl = pl.ds(cc * 16, 16)
                        yb[r, sl] = 0.8 * yb[r, sl] + ab[r, sl] * accb[r, sl]
                    return carry

                lax.fori_loop(0, CHUNK, upd, 0)
                pltpu.sync_copy(yb, y_out.at[pl.ds(r0, CHUNK)])


def _sc_degree(dst_g, ones_w, zeros_w8):
    fn = pl.kernel(
        _deg_body,
        jax.ShapeDtypeStruct((T * NPAD, 8), f32),
        mesh=_MESH,
        scratch_types=[
            pltpu.VMEM((NW, WIN), i32),
            pltpu.VMEM((WIN, 8), f32),
            pltpu.VMEM_SHARED((NPAD, 8), f32),
        ],
        compiler_params=_SC_PARAMS,
    )
    return fn(dst_g, ones_w, zeros_w8)


def _sc_diffusion(y0, a_full, src_g, dst_g, zeros_w64):
    fn = pl.kernel(
        _diff_body,
        jax.ShapeDtypeStruct((T * NPAD, D), f32),
        mesh=_MESH,
        scratch_types=[
            pltpu.VMEM((NW, WIN), i32),
            pltpu.VMEM((NW, WIN), i32),
            pltpu.VMEM((WIN, D), f32),
            pltpu.VMEM((WIN, D), f32),
            pltpu.VMEM((CHUNK, D), f32),
            pltpu.VMEM((CHUNK, D), f32),
            pltpu.VMEM((CHUNK, D), f32),
            pltpu.VMEM_SHARED((NPAD, D), f32),
            pltpu.SemaphoreType.DMA,
            pltpu.SemaphoreType.DMA,
        ],
        compiler_params=_SC_PARAMS,
    )
    return fn(y0, a_full, src_g, dst_g, zeros_w64)


RBLK = 512


def _p1_body(nf_ref, deg_ref, wi_ref, bi_ref, w0_ref, b0_ref,
             y0_ref, a_ref, nrm_ref, invn_ref):
    deg = jnp.maximum(deg_ref[:, 0:1], 1.0)
    nrm = lax.rsqrt(deg)
    x0 = jnp.dot(nf_ref[...], wi_ref[...],
                 preferred_element_type=f32) + bi_ref[...]
    h = jnp.dot(x0, w0_ref[...], preferred_element_type=f32) + b0_ref[...]
    y0_ref[...] = h * nrm
    a_ref[...] = jnp.broadcast_to(DT * nrm * nrm, (RBLK, D))
    nrm_ref[...] = jnp.broadcast_to(nrm, (RBLK, D))
    invn_ref[...] = jnp.broadcast_to(1.0 / nrm, (RBLK, D))


def _p2_body(y5_ref, nrm_ref, invn_ref, w1_ref, b1_ref, yo_ref):
    x1 = jnp.maximum(y5_ref[...], 0.0) * invn_ref[...]
    h = jnp.dot(x1, w1_ref[...], preferred_element_type=f32) + b1_ref[...]
    yo_ref[...] = h * nrm_ref[...]


def _attn_body(y_ref, invn_ref, te_ref, wk_ref, wv_ref, qf_ref, sel_ref,
               selt_ref, wo_ref, bo_ref, wh1_ref, bh1_ref, wh2_ref, bh2_ref,
               o_ref):
    scale = 1.0 / math.sqrt(DH)
    ss = []
    vs = []
    for t in range(T):
        x_t = jnp.maximum(y_ref[t], 0.0) * invn_ref[t]
        h_t = x_t + te_ref[t:t + 1, :]
        k_t = jnp.dot(h_t, wk_ref[...], preferred_element_type=f32)
        v_t = jnp.dot(h_t, wv_ref[...], preferred_element_type=f32)
        s_t = jnp.dot(k_t * qf_ref[...], sel_ref[...],
                      preferred_element_type=f32) * scale
        ss.append(s_t)
        vs.append(v_t)
    m = jnp.maximum(jnp.maximum(ss[0], ss[1]), jnp.maximum(ss[2], ss[3]))
    es = [jnp.exp(s_t - m) for s_t in ss]
    z = es[0] + es[1] + es[2] + es[3]
    pooled = jnp.zeros_like(vs[0])
    for t in range(T):
        a_t = jnp.dot(es[t] / z, selt_ref[...], preferred_element_type=f32)
        pooled = pooled + a_t * vs[t]
    out = jnp.dot(pooled, wo_ref[...], preferred_element_type=f32) + bo_ref[...]
    hid = jnp.maximum(
        jnp.dot(out, wh1_ref[...], preferred_element_type=f32) + bh1_ref[...],
        0.0)
    o_ref[...] = jnp.dot(hid, wh2_ref[...],
                         preferred_element_type=f32) + bh2_ref[...]


def _tc_prep1(nf_pad, deg8, W_in, b_in, W0, b0):
    nb = (T * NPAD) // RBLK
    full = lambda shape: pl.BlockSpec(shape, lambda i: (0, 0))
    return pl.pallas_call(
        _p1_body,
        grid=(nb,),
        in_specs=[
            pl.BlockSpec((RBLK, F_IN), lambda i: (i, 0)),
            pl.BlockSpec((RBLK, 8), lambda i: (i, 0)),
            full((F_IN, D)), full((1, D)), full((D, D)), full((1, D)),
        ],
        out_specs=[pl.BlockSpec((RBLK, D), lambda i: (i, 0))] * 4,
        out_shape=[jax.ShapeDtypeStruct((T * NPAD, D), f32)] * 4,
    )(nf_pad, deg8, W_in, b_in, W0, b0)


def _tc_prep2(y5, nrm_e, invn_e, W1, b1):
    nb = (T * NPAD) // RBLK
    full = lambda shape: pl.BlockSpec(shape, lambda i: (0, 0))
    return pl.pallas_call(
        _p2_body,
        grid=(nb,),
        in_specs=[pl.BlockSpec((RBLK, D), lambda i: (i, 0))] * 3 +
                 [full((D, D)), full((1, D))],
        out_specs=pl.BlockSpec((RBLK, D), lambda i: (i, 0)),
        out_shape=jax.ShapeDtypeStruct((T * NPAD, D), f32),
    )(y5, nrm_e, invn_e, W1, b1)


def _tc_attn(y5l, invn_e, te, W_k, W_v, qf, sel, selt, W_o, b_o,
             W_h1, b_h1, W_h2p, b_h2p):
    nbb = NPAD // RBLK
    full = lambda shape: pl.BlockSpec(shape, lambda i: (0, 0))
    y4 = y5l.reshape(T, NPAD, D)
    i4 = invn_e.reshape(T, NPAD, D)
    return pl.pallas_call(
        _attn_body,
        grid=(nbb,),
        in_specs=[
            pl.BlockSpec((T, RBLK, D), lambda i: (0, i, 0)),
            pl.BlockSpec((T, RBLK, D), lambda i: (0, i, 0)),
            full((T, D)), full((D, D)), full((D, D)), full((1, D)),
            full((D, H)), full((H, D)), full((D, D)), full((1, D)),
            full((D, D // 2)), full((1, D // 2)),
            full((D // 2, 128)), full((1, 128)),
        ],
        out_specs=pl.BlockSpec((RBLK, 128), lambda i: (i, 0)),
        out_shape=jax.ShapeDtypeStruct((NPAD, 128), f32),
    )(y4, i4, te, W_k, W_v, qf, sel, selt, W_o, b_o, W_h1, b_h1, W_h2p, b_h2p)


def kernel(node_features, edge_index, times, W_in, b_in, W_diff, b_diff,
           t2v_w0, t2v_b0, t2v_w, t2v_b, W_time, b_time, W_k, W_v, q_attn,
           W_o, b_o, W_h1, b_h1, W_h2, b_h2):
    epad = 16 * EPT - E
    pad_rows = (N + (jnp.arange(epad, dtype=i32) % NDUMP)).astype(i32)
    src_slabs = []
    dst_slabs = []
    for t in range(T):
        s_t = jnp.concatenate([edge_index[t, 0], pad_rows]) + t * NPAD
        d_t = jnp.concatenate([edge_index[t, 1], pad_rows])
        src_slabs.append(s_t.reshape(16, NW, WIN))
        dst_slabs.append(d_t.reshape(16, NW, WIN))
    src_g = jnp.stack(src_slabs).reshape(T * 16, NW, WIN).astype(i32)
    dst_g = jnp.stack(dst_slabs).reshape(T * 16, NW, WIN).astype(i32)

    nf_pad = jnp.pad(node_features, ((0, 0), (0, NPAD - N), (0, 0)))
    nf_pad = nf_pad.reshape(T * NPAD, F_IN)

    ones_w = jnp.ones((WIN, 8), f32)
    zeros_w8 = jnp.zeros((RPT, 8), f32)
    zeros_w64 = jnp.zeros((RPT, D), f32)

    tv = jnp.concatenate([times[:, None] * t2v_w0 + t2v_b0,
                          jnp.sin(times[:, None] * t2v_w + t2v_b)], axis=1)
    te = tv @ W_time + b_time

    qf = q_attn.reshape(1, D)
    sel = jnp.repeat(jnp.eye(H, dtype=f32), DH, axis=0)
    selt = sel.T
    W_h2p = jnp.pad(W_h2, ((0, 0), (0, 128 - W_h2.shape[1])))
    b_h2p = jnp.pad(b_h2, (0, 128 - b_h2.shape[0]))

    deg8 = _sc_degree(dst_g, ones_w, zeros_w8)

    y0, a_full, nrm_e, invn_e = _tc_prep1(
        nf_pad, deg8, W_in, b_in.reshape(1, D),
        W_diff[0], b_diff[0].reshape(1, D))

    y5 = _sc_diffusion(y0, a_full, src_g, dst_g, zeros_w64)
    y0b = _tc_prep2(y5, nrm_e, invn_e, W_diff[1], b_diff[1].reshape(1, D))
    y5b = _sc_diffusion(y0b, a_full, src_g, dst_g, zeros_w64)

    logits_pad = _tc_attn(y5b, invn_e, te, W_k, W_v, qf, sel, selt,
                          W_o, b_o.reshape(1, D), W_h1, b_h1.reshape(1, D // 2),
                          W_h2p, b_h2p.reshape(1, 128))
    return logits_pad[:N, :2]


# speedup vs baseline: 13.6031x; 1.2729x over previous
import functools
import math

import jax
import jax.numpy as jnp
from jax import lax
from jax.experimental import pallas as pl
from jax.experimental.pallas import tpu as pltpu
from jax.experimental.pallas import tpu_sc as plsc

N = 10000
E = 320000
T = 4
F_IN = 128
D = 64
TD = 16
H = 4
DH = 16
STEPS = 5
DT = 0.2

NPAD = 10240
RPT = NPAD // 16
CHUNK = 64
WIN = 128
NW = 160
EPT = NW * WIN
NDUMP = NPAD - N

f32 = jnp.float32
i32 = jnp.int32

_MESH = plsc.VectorSubcoreMesh(
    core_axis_name="c", subcore_axis_name="s", num_cores=2, num_subcores=16)
_SC_PARAMS = pltpu.CompilerParams(use_tc_tiling_on_sc=False)


def _deg_body(dst_hbm, ones_hbm, zeros_hbm, deg_out, dstv, onesv, degacc):
    c = lax.axis_index("c")
    s = lax.axis_index("s")
    pltpu.sync_copy(ones_hbm, onesv)
    for si in range(2):
        t = 2 * c + si
        w = t * 16 + s
        pltpu.sync_copy(dst_hbm.at[w], dstv)
        pltpu.sync_copy(zeros_hbm, degacc.at[pl.ds(s * RPT, RPT)])
        plsc.subcore_barrier()

        def win(j, carry):
            pltpu.sync_copy(onesv, degacc.at[dstv.at[j]], add=True)
            return carry

        lax.fori_loop(0, NW, win, 0)
        plsc.subcore_barrier()
        pltpu.sync_copy(degacc.at[pl.ds(s * RPT, RPT)],
                        deg_out.at[pl.ds(t * NPAD + s * RPT, RPT)])
        plsc.subcore_barrier()


def _diff_body(y_in, a_hbm, src_hbm, dst_hbm, zeros_hbm, y_out,
               srcv, dstv, gb, accb, yb, ab, acc, gsem, ssem):
    c = lax.axis_index("c")
    s = lax.axis_index("s")
    NB = len(gb)
    for si in range(2):
        t = 2 * c + si
        w = t * 16 + s
        pltpu.sync_copy(src_hbm.at[w], srcv)
        pltpu.sync_copy(dst_hbm.at[w], dstv)
        if si == 0:
            pltpu.sync_copy(zeros_hbm, acc.at[pl.ds(s * RPT, RPT)])
        base = t * NPAD + s * RPT
        for p in range(STEPS):
            ytab = y_in if p == 0 else y_out
            plsc.subcore_barrier()

            for b in range(NB):
                pltpu.async_copy(ytab.at[srcv.at[b]], gb[b], gsem[b])

            def rnd(i, carry, ytab=ytab):
                j0 = NB * i
                for b in range(NB):
                    pltpu.make_async_copy(ytab.at[srcv.at[0]],
                                          gb[b], gsem[b]).wait()
                    pltpu.async_copy(gb[b], acc.at[dstv.at[j0 + b]],
                                     ssem[b], add=True)
                for b in range(NB):
                    @pl.when(j0 + b + NB < NW)
                    def _(b=b):
                        pltpu.make_async_copy(gb[b], acc.at[dstv.at[0]],
                                              ssem[b]).wait()
                        pltpu.async_copy(ytab.at[srcv.at[j0 + b + NB]],
                                         gb[b], gsem[b])
                return carry

            lax.fori_loop(0, NW // NB, rnd, 0)
            for b in range(NB):
                pltpu.make_async_copy(gb[b], acc.at[dstv.at[0]],
                                      ssem[b]).wait()
            plsc.subcore_barrier()
            for k in range(RPT // CHUNK):
                l0 = s * RPT + k * CHUNK
                r0 = base + k * CHUNK
                pltpu.sync_copy(acc.at[pl.ds(l0, CHUNK)], accb)
                pltpu.sync_copy(zeros_hbm.at[pl.ds(0, CHUNK)],
                                acc.at[pl.ds(l0, CHUNK)])
                pltpu.sync_copy(ytab.at[pl.ds(r0, CHUNK)], yb)
                pltpu.sync_copy(a_hbm.at[pl.ds(r0, CHUNK)], ab)

                def upd(r, carry):
                    for cc in range(4):
                        sl = pl.ds(cc * 16, 16)
                        yb[r, sl] = 0.8 * yb[r, sl] + ab[r, sl] * accb[r, sl]
                    return carry

                lax.fori_loop(0, CHUNK, upd, 0)
                pltpu.sync_copy(yb, y_out.at[pl.ds(r0, CHUNK)])


def _sc_degree(dst_g, ones_w, zeros_w8):
    fn = pl.kernel(
        _deg_body,
        jax.ShapeDtypeStruct((T * NPAD, 8), f32),
        mesh=_MESH,
        scratch_types=[
            pltpu.VMEM((NW, WIN), i32),
            pltpu.VMEM((WIN, 8), f32),
            pltpu.VMEM_SHARED((NPAD, 8), f32),
        ],
        compiler_params=_SC_PARAMS,
    )
    return fn(dst_g, ones_w, zeros_w8)


def _sc_diffusion(y0, a_full, src_g, dst_g, zeros_w64):
    fn = pl.kernel(
        _diff_body,
        jax.ShapeDtypeStruct((T * NPAD, D), f32),
        mesh=_MESH,
        scratch_types=[
            pltpu.VMEM((NW, WIN), i32),
            pltpu.VMEM((NW, WIN), i32),
            [pltpu.VMEM((WIN, D), f32) for _ in range(4)],
            pltpu.VMEM((CHUNK, D), f32),
            pltpu.VMEM((CHUNK, D), f32),
            pltpu.VMEM((CHUNK, D), f32),
            pltpu.VMEM_SHARED((NPAD, D), f32),
            [pltpu.SemaphoreType.DMA for _ in range(4)],
            [pltpu.SemaphoreType.DMA for _ in range(4)],
        ],
        compiler_params=_SC_PARAMS,
    )
    return fn(y0, a_full, src_g, dst_g, zeros_w64)


RBLK = 512


def _p1_body(nf_ref, deg_ref, wi_ref, bi_ref, w0_ref, b0_ref,
             y0_ref, a_ref, nrm_ref, invn_ref):
    deg = jnp.maximum(deg_ref[:, 0:1], 1.0)
    nrm = lax.rsqrt(deg)
    x0 = jnp.dot(nf_ref[...], wi_ref[...],
                 preferred_element_type=f32) + bi_ref[...]
    h = jnp.dot(x0, w0_ref[...], preferred_element_type=f32) + b0_ref[...]
    y0_ref[...] = h * nrm
    a_ref[...] = jnp.broadcast_to(DT * nrm * nrm, (RBLK, D))
    nrm_ref[...] = jnp.broadcast_to(nrm, (RBLK, D))
    invn_ref[...] = jnp.broadcast_to(1.0 / nrm, (RBLK, D))


def _p2_body(y5_ref, nrm_ref, invn_ref, w1_ref, b1_ref, yo_ref):
    x1 = jnp.maximum(y5_ref[...], 0.0) * invn_ref[...]
    h = jnp.dot(x1, w1_ref[...], preferred_element_type=f32) + b1_ref[...]
    yo_ref[...] = h * nrm_ref[...]


def _attn_body(y_ref, invn_ref, te_ref, wk_ref, wv_ref, qf_ref, sel_ref,
               selt_ref, wo_ref, bo_ref, wh1_ref, bh1_ref, wh2_ref, bh2_ref,
               o_ref):
    scale = 1.0 / math.sqrt(DH)
    ss = []
    vs = []
    for t in range(T):
        x_t = jnp.maximum(y_ref[t], 0.0) * invn_ref[t]
        h_t = x_t + te_ref[t:t + 1, :]
        k_t = jnp.dot(h_t, wk_ref[...], preferred_element_type=f32)
        v_t = jnp.dot(h_t, wv_ref[...], preferred_element_type=f32)
        s_t = jnp.dot(k_t * qf_ref[...], sel_ref[...],
                      preferred_element_type=f32) * scale
        ss.append(s_t)
        vs.append(v_t)
    m = jnp.maximum(jnp.maximum(ss[0], ss[1]), jnp.maximum(ss[2], ss[3]))
    es = [jnp.exp(s_t - m) for s_t in ss]
    z = es[0] + es[1] + es[2] + es[3]
    pooled = jnp.zeros_like(vs[0])
    for t in range(T):
        a_t = jnp.dot(es[t] / z, selt_ref[...], preferred_element_type=f32)
        pooled = pooled + a_t * vs[t]
    out = jnp.dot(pooled, wo_ref[...], preferred_element_type=f32) + bo_ref[...]
    hid = jnp.maximum(
        jnp.dot(out, wh1_ref[...], preferred_element_type=f32) + bh1_ref[...],
        0.0)
    o_ref[...] = jnp.dot(hid, wh2_ref[...],
                         preferred_element_type=f32) + bh2_ref[...]


def _tc_prep1(nf_pad, deg8, W_in, b_in, W0, b0):
    nb = (T * NPAD) // RBLK
    full = lambda shape: pl.BlockSpec(shape, lambda i: (0, 0))
    return pl.pallas_call(
        _p1_body,
        grid=(nb,),
        in_specs=[
            pl.BlockSpec((RBLK, F_IN), lambda i: (i, 0)),
            pl.BlockSpec((RBLK, 8), lambda i: (i, 0)),
            full((F_IN, D)), full((1, D)), full((D, D)), full((1, D)),
        ],
        out_specs=[pl.BlockSpec((RBLK, D), lambda i: (i, 0))] * 4,
        out_shape=[jax.ShapeDtypeStruct((T * NPAD, D), f32)] * 4,
    )(nf_pad, deg8, W_in, b_in, W0, b0)


def _tc_prep2(y5, nrm_e, invn_e, W1, b1):
    nb = (T * NPAD) // RBLK
    full = lambda shape: pl.BlockSpec(shape, lambda i: (0, 0))
    return pl.pallas_call(
        _p2_body,
        grid=(nb,),
        in_specs=[pl.BlockSpec((RBLK, D), lambda i: (i, 0))] * 3 +
                 [full((D, D)), full((1, D))],
        out_specs=pl.BlockSpec((RBLK, D), lambda i: (i, 0)),
        out_shape=jax.ShapeDtypeStruct((T * NPAD, D), f32),
    )(y5, nrm_e, invn_e, W1, b1)


def _tc_attn(y5l, invn_e, te, W_k, W_v, qf, sel, selt, W_o, b_o,
             W_h1, b_h1, W_h2p, b_h2p):
    nbb = NPAD // RBLK
    full = lambda shape: pl.BlockSpec(shape, lambda i: (0, 0))
    y4 = y5l.reshape(T, NPAD, D)
    i4 = invn_e.reshape(T, NPAD, D)
    return pl.pallas_call(
        _attn_body,
        grid=(nbb,),
        in_specs=[
            pl.BlockSpec((T, RBLK, D), lambda i: (0, i, 0)),
            pl.BlockSpec((T, RBLK, D), lambda i: (0, i, 0)),
            full((T, D)), full((D, D)), full((D, D)), full((1, D)),
            full((D, H)), full((H, D)), full((D, D)), full((1, D)),
            full((D, D // 2)), full((1, D // 2)),
            full((D // 2, 128)), full((1, 128)),
        ],
        out_specs=pl.BlockSpec((RBLK, 128), lambda i: (i, 0)),
        out_shape=jax.ShapeDtypeStruct((NPAD, 128), f32),
    )(y4, i4, te, W_k, W_v, qf, sel, selt, W_o, b_o, W_h1, b_h1, W_h2p, b_h2p)


def kernel(node_features, edge_index, times, W_in, b_in, W_diff, b_diff,
           t2v_w0, t2v_b0, t2v_w, t2v_b, W_time, b_time, W_k, W_v, q_attn,
           W_o, b_o, W_h1, b_h1, W_h2, b_h2):
    epad = 16 * EPT - E
    pad_rows = (N + (jnp.arange(epad, dtype=i32) % NDUMP)).astype(i32)
    src_slabs = []
    dst_slabs = []
    for t in range(T):
        s_t = jnp.concatenate([edge_index[t, 0], pad_rows]) + t * NPAD
        d_t = jnp.concatenate([edge_index[t, 1], pad_rows])
        src_slabs.append(s_t.reshape(16, NW, WIN))
        dst_slabs.append(d_t.reshape(16, NW, WIN))
    src_g = jnp.stack(src_slabs).reshape(T * 16, NW, WIN).astype(i32)
    dst_g = jnp.stack(dst_slabs).reshape(T * 16, NW, WIN).astype(i32)

    nf_pad = jnp.pad(node_features, ((0, 0), (0, NPAD - N), (0, 0)))
    nf_pad = nf_pad.reshape(T * NPAD, F_IN)

    ones_w = jnp.ones((WIN, 8), f32)
    zeros_w8 = jnp.zeros((RPT, 8), f32)
    zeros_w64 = jnp.zeros((RPT, D), f32)

    tv = jnp.concatenate([times[:, None] * t2v_w0 + t2v_b0,
                          jnp.sin(times[:, None] * t2v_w + t2v_b)], axis=1)
    te = tv @ W_time + b_time

    qf = q_attn.reshape(1, D)
    sel = jnp.repeat(jnp.eye(H, dtype=f32), DH, axis=0)
    selt = sel.T
    W_h2p = jnp.pad(W_h2, ((0, 0), (0, 128 - W_h2.shape[1])))
    b_h2p = jnp.pad(b_h2, (0, 128 - b_h2.shape[0]))

    deg8 = _sc_degree(dst_g, ones_w, zeros_w8)

    y0, a_full, nrm_e, invn_e = _tc_prep1(
        nf_pad, deg8, W_in, b_in.reshape(1, D),
        W_diff[0], b_diff[0].reshape(1, D))

    y5 = _sc_diffusion(y0, a_full, src_g, dst_g, zeros_w64)
    y0b = _tc_prep2(y5, nrm_e, invn_e, W_diff[1], b_diff[1].reshape(1, D))
    y5b = _sc_diffusion(y0b, a_full, src_g, dst_g, zeros_w64)

    logits_pad = _tc_attn(y5b, invn_e, te, W_k, W_v, qf, sel, selt,
                          W_o, b_o.reshape(1, D), W_h1, b_h1.reshape(1, D // 2),
                          W_h2p, b_h2p.reshape(1, 128))
    return logits_pad[:N, :2]
